# baseline (speedup 1.0000x reference)
import functools

import jax
import jax.numpy as jnp
from jax import lax
from jax.experimental import pallas as pl
from jax.experimental.pallas import tpu as pltpu
from jax.experimental.pallas import tpu_sc as plsc

NC = 2
NS = 16
NW = NC * NS
L = 16
CHUNK = 128
NBUF = 2
M_STREAM = 96


@functools.cache
def _build(B, V, D):
    b_per_w = B // NW
    n_chunks = b_per_w // CHUNK
    n_pairs = n_chunks // NBUF
    kcols = D // L
    mesh = plsc.VectorSubcoreMesh(core_axis_name="c", subcore_axis_name="s")

    @functools.partial(
        pl.kernel,
        out_type=jax.ShapeDtypeStruct((B, D), jnp.float32),
        mesh=mesh,
        compiler_params=pltpu.CompilerParams(needs_layout_passes=False),
        scratch_types=[
            pltpu.VMEM((n_chunks, CHUNK), jnp.int32),
            pltpu.VMEM((V * D,), jnp.float32),
            pltpu.VMEM((NBUF, CHUNK, D), jnp.float32),
            pltpu.SemaphoreType.DMA((NBUF,)),
            pltpu.SemaphoreType.DMA((NBUF,)),
        ],
    )
    def gather_kernel(idx_hbm, table_hbm, tablef_hbm, out_hbm, idx_v,
                      table_v, rows_v, gsem, wsem):
        wid = lax.axis_index("s") * NC + lax.axis_index("c")
        base = wid * b_per_w

        pltpu.sync_copy(idx_hbm.at[wid], idx_v)
        pltpu.sync_copy(tablef_hbm, table_v)

        iota = lax.iota(jnp.int32, L)
        ones = jnp.ones((L,), jnp.bool_)
        dnums = lax.GatherDimensionNumbers(
            offset_dims=(), collapsed_slice_dims=(0,), start_index_map=(0,))

        def lane_broadcast(vec, rr):
            return lax.gather(
                vec, jnp.full((L, 1), rr, jnp.int32), dnums, (1,),
                mode=lax.GatherScatterMode.PROMISE_IN_BOUNDS)

        def gather(j, buf):
            return pltpu.make_async_copy(
                table_hbm.at[idx_v.at[j, pl.ds(0, M_STREAM)]],
                rows_v.at[buf, pl.ds(0, M_STREAM)],
                gsem.at[buf])

        def write(j, buf):
            return pltpu.make_async_copy(
                rows_v.at[buf],
                out_hbm.at[pl.ds(base + j * CHUNK, CHUNK)],
                wsem.at[buf])

        def build_rows(j, buf):
            def group(g, carry):
                row0 = M_STREAM + g * L
                idv = idx_v[j, pl.ds(row0, L)]
                addrs = idv * D
                for rr in range(L):
                    addr = lane_broadcast(addrs, rr)
                    for k in range(kcols):
                        cidx = addr + (iota + k * L)
                        vals = plsc.load_gather(table_v, [cidx])
                        plsc.store_compressed(
                            rows_v.at[buf, row0 + rr, pl.ds(k * L, L)],
                            vals, mask=ones)
                return carry

            lax.fori_loop(0, (CHUNK - M_STREAM) // L, group, 0)

        def body(p, carry):
            for buf in range(NBUF):
                j = p * NBUF + buf

                @pl.when(j >= NBUF)
                def _():
                    write(j - NBUF, buf).wait()

                gather(j, buf).start()
                build_rows(j, buf)
                gather(j, buf).wait()
                write(j, buf).start()
            return carry

        lax.fori_loop(0, n_pairs, body, 0)

        for j in range(n_chunks - NBUF, n_chunks):
            write(j, j % NBUF).wait()

    return gather_kernel


def kernel(timesteps, embeddings):
    B = timesteps.size
    V, D = embeddings.shape
    idx = timesteps.reshape(NW, B // (NW * CHUNK), CHUNK)
    out = _build(B, V, D)(idx, embeddings, embeddings.reshape(V * D))
    return out.reshape(*timesteps.shape, D)


# speedup vs baseline: 1.4064x; 1.4064x over previous
import functools

import jax
import jax.numpy as jnp
from jax import lax
from jax.experimental import pallas as pl
from jax.experimental.pallas import tpu as pltpu
from jax.experimental.pallas import tpu_sc as plsc

NC = 2
NS = 16
NW = NC * NS
L = 16
CHUNK = 128
NBUF = 2
M_STREAM = 48


@functools.cache
def _build(B, V, D):
    b_per_w = B // NW
    n_chunks = b_per_w // CHUNK
    n_pairs = n_chunks // NBUF
    kcols = D // L
    mesh = plsc.VectorSubcoreMesh(core_axis_name="c", subcore_axis_name="s")

    @functools.partial(
        pl.kernel,
        out_type=jax.ShapeDtypeStruct((B, D), jnp.float32),
        mesh=mesh,
        compiler_params=pltpu.CompilerParams(needs_layout_passes=False),
        scratch_types=[
            pltpu.VMEM((n_chunks, CHUNK), jnp.int32),
            pltpu.VMEM((V * D,), jnp.float32),
            pltpu.VMEM((NBUF, CHUNK, D), jnp.float32),
            pltpu.SemaphoreType.DMA((NBUF,)),
            pltpu.SemaphoreType.DMA((NBUF,)),
        ],
    )
    def gather_kernel(idx_hbm, table_hbm, tablef_hbm, out_hbm, idx_v,
                      table_v, rows_v, gsem, wsem):
        wid = lax.axis_index("s") * NC + lax.axis_index("c")
        base = wid * b_per_w

        pltpu.sync_copy(idx_hbm.at[wid], idx_v)
        pltpu.sync_copy(tablef_hbm, table_v)

        iota = lax.iota(jnp.int32, L)
        ones = jnp.ones((L,), jnp.bool_)
        dnums = lax.GatherDimensionNumbers(
            offset_dims=(), collapsed_slice_dims=(0,), start_index_map=(0,))

        def lane_broadcast(vec, rr):
            return lax.gather(
                vec, jnp.full((L, 1), rr, jnp.int32), dnums, (1,),
                mode=lax.GatherScatterMode.PROMISE_IN_BOUNDS)

        def gather(j, buf):
            return pltpu.make_async_copy(
                table_hbm.at[idx_v.at[j, pl.ds(0, M_STREAM)]],
                rows_v.at[buf, pl.ds(0, M_STREAM)],
                gsem.at[buf])

        def write(j, buf):
            return pltpu.make_async_copy(
                rows_v.at[buf],
                out_hbm.at[pl.ds(base + j * CHUNK, CHUNK)],
                wsem.at[buf])

        def build_rows(j, buf):
            def group(g, carry):
                row0 = M_STREAM + g * L
                idv = idx_v[j, pl.ds(row0, L)]
                addrs = idv * D
                for rr in range(L):
                    addr = lane_broadcast(addrs, rr)
                    for k in range(kcols):
                        cidx = addr + (iota + k * L)
                        vals = plsc.load_gather(table_v, [cidx])
                        plsc.store_compressed(
                            rows_v.at[buf, row0 + rr, pl.ds(k * L, L)],
                            vals, mask=ones)
                return carry

            lax.fori_loop(0, (CHUNK - M_STREAM) // L, group, 0)

        def body(p, carry):
            for buf in range(NBUF):
                j = p * NBUF + buf

                @pl.when(j >= NBUF)
                def _():
                    write(j - NBUF, buf).wait()

                gather(j, buf).start()
                build_rows(j, buf)
                gather(j, buf).wait()
                write(j, buf).start()
            return carry

        lax.fori_loop(0, n_pairs, body, 0)

        for j in range(n_chunks - NBUF, n_chunks):
            write(j, j % NBUF).wait()

    return gather_kernel


def kernel(timesteps, embeddings):
    B = timesteps.size
    V, D = embeddings.shape
    idx = timesteps.reshape(NW, B // (NW * CHUNK), CHUNK)
    out = _build(B, V, D)(idx, embeddings, embeddings.reshape(V * D))
    return out.reshape(*timesteps.shape, D)
